# baseline (device time: 194611 ns/iter reference)
import jax
import jax.numpy as jnp
from jax import lax
from jax.experimental import pallas as pl
from jax.experimental.pallas import tpu as pltpu

N_DEV = 4
SQ = 1024
SKV_LOCAL = 1024
SKV = N_DEV * SKV_LOCAL
HQ_LOCAL = 8
DH = 128
DMODEL = 1024
SCALE = 0.08838834764831843
NEG = jnp.float32(-1e30)


def kernel(x, Wq, K_ext, V_ext, Wo):
    xb = x[0].astype(jnp.bfloat16)
    wqb = Wq.astype(jnp.bfloat16)
    kb = K_ext[0].astype(jnp.bfloat16)
    vb = V_ext[0].astype(jnp.bfloat16)
    wob = Wo.astype(jnp.bfloat16)

    def body(x_ref, wq_ref, k_ref, v_ref, wo_ref, out_ref,
             k_all, v_all, p_all, q_ref, ctx_ref, m_ref, l_ref,
             k_send, v_send, p_send, k_recv, v_recv, p_recv, local_sem):
        my = lax.axis_index("i")

        k_local = pltpu.make_async_copy(
            k_ref.at[:, pl.ds(my * HQ_LOCAL, HQ_LOCAL), :],
            k_all.at[0], local_sem.at[0])
        k_local.start()
        v_local = pltpu.make_async_copy(
            v_ref.at[:, pl.ds(my * HQ_LOCAL, HQ_LOCAL), :],
            v_all.at[0], local_sem.at[1])
        v_local.start()

        barrier = pltpu.get_barrier_semaphore()
        for o in range(1, N_DEV):
            peer = lax.rem(my + o, N_DEV)
            pl.semaphore_signal(barrier, inc=1, device_id=(peer,),
                                device_id_type=pl.DeviceIdType.MESH)
        pl.semaphore_wait(barrier, N_DEV - 1)

        kv_pairs = ((k_ref, k_all, k_send, k_recv),
                    (v_ref, v_all, v_send, v_recv))

        def issue_kv(o):
            t = lax.rem(my + o, N_DEV)
            out = []
            for (src, all_, ssem, rsem) in kv_pairs:
                rdma = pltpu.make_async_remote_copy(
                    src_ref=src.at[:, pl.ds(t * HQ_LOCAL, HQ_LOCAL), :],
                    dst_ref=all_.at[4 - o],
                    send_sem=ssem.at[o],
                    recv_sem=rsem.at[4 - o],
                    device_id=(t,),
                    device_id_type=pl.DeviceIdType.MESH,
                )
                rdma.start()
                out.append(rdma)
            return out

        def wait_kv(sig):
            s = lax.rem(my + sig, N_DEV)
            for (src, all_, ssem, rsem) in kv_pairs:
                recv = pltpu.make_async_remote_copy(
                    src_ref=src.at[:, pl.ds(0, HQ_LOCAL), :],
                    dst_ref=all_.at[sig],
                    send_sem=ssem.at[sig],
                    recv_sem=rsem.at[sig],
                    device_id=(s,),
                    device_id_type=pl.DeviceIdType.MESH,
                )
                recv.wait_recv()

        nbr_sends = issue_kv(1) + issue_kv(3)

        q_ref[...] = (lax.dot(x_ref[...], wq_ref[...],
                              preferred_element_type=jnp.float32)
                      * SCALE).astype(jnp.bfloat16)

        def group_slice(all_, sig, h, g):
            a = all_[sig, :, h, :].reshape(4, 4, 64, DH)
            return a[:, g].reshape(256, DH)

        def flash_phase(sigs, first):
            for g in range(4):
                for h in range(HQ_LOCAL):
                    gh = g * HQ_LOCAL + h
                    qg = q_ref[:, h * DH:(h + 1) * DH].reshape(
                        4, 4, 64, DH)[:, g].reshape(256, DH)
                    kg = jnp.concatenate(
                        [group_slice(k_all, sig, h, g) for sig in sigs], 0)
                    vg = jnp.concatenate(
                        [group_slice(v_all, sig, h, g) for sig in sigs], 0)
                    s_ = lax.dot_general(qg, kg, (((1,), (1,)), ((), ())),
                                         preferred_element_type=jnp.float32)
                    m2 = jnp.max(s_, axis=1, keepdims=True)
                    if first:
                        mn = m2
                        w = jnp.exp(s_ - mn)
                        ln = jnp.sum(w, axis=1, keepdims=True)
                        acc = lax.dot_general(
                            w.astype(jnp.bfloat16), vg,
                            (((1,), (0,)), ((), ())),
                            preferred_element_type=jnp.float32)
                    else:
                        m1 = m_ref[gh]
                        l1 = l_ref[gh]
                        mn = jnp.maximum(m1, m2)
                        a1 = jnp.exp(m1 - mn) * l1
                        w = jnp.exp(s_ - mn)
                        ln = a1 + jnp.sum(w, axis=1, keepdims=True)
                        prev = ctx_ref[pl.ds(256 * g, 256),
                                       pl.ds(h * DH, DH)].astype(jnp.float32)
                        acc = prev * a1 + lax.dot_general(
                            w.astype(jnp.bfloat16), vg,
                            (((1,), (0,)), ((), ())),
                            preferred_element_type=jnp.float32)
                    m_ref[gh] = mn
                    l_ref[gh] = ln
                    ctx_ref[pl.ds(256 * g, 256), pl.ds(h * DH, DH)] = (
                        acc / ln).astype(jnp.bfloat16)

        k_local.wait()
        v_local.wait()
        flash_phase((0,), first=True)

        for r in nbr_sends:
            r.wait_send()
        sends = issue_kv(2)

        wait_kv(1)
        wait_kv(3)
        flash_phase((1, 3), first=False)

        wait_kv(2)
        flash_phase((2,), first=False)

        for g in range(4):
            p_all[pl.ds(0, 1), pl.ds(256 * g, 256), :] = lax.dot(
                ctx_ref[pl.ds(256 * g, 256), :], wo_ref[...],
                preferred_element_type=jnp.float32).astype(jnp.bfloat16)[None]
            for o in range(1, N_DEV):
                t = lax.rem(my + o, N_DEV)
                rdma = pltpu.make_async_remote_copy(
                    src_ref=p_all.at[0, pl.ds(256 * g, 256)],
                    dst_ref=p_all.at[4 - o, pl.ds(256 * g, 256)],
                    send_sem=p_send.at[o, g],
                    recv_sem=p_recv.at[4 - o, g],
                    device_id=(t,),
                    device_id_type=pl.DeviceIdType.MESH,
                )
                rdma.start()
                sends.append(rdma)

        for sig in (1, 3, 2):
            s = lax.rem(my + sig, N_DEV)
            for g in range(4):
                recv = pltpu.make_async_remote_copy(
                    src_ref=p_all.at[0, pl.ds(256 * g, 256)],
                    dst_ref=p_all.at[sig, pl.ds(256 * g, 256)],
                    send_sem=p_send.at[sig, g],
                    recv_sem=p_recv.at[sig, g],
                    device_id=(s,),
                    device_id_type=pl.DeviceIdType.MESH,
                )
                recv.wait_recv()

        for c in range(4):
            for g in range(4):
                src_rows = pl.ds(256 * g + 64 * c, 64)
                acc = (p_all[0, src_rows, :].astype(jnp.float32)
                       + p_all[1, src_rows, :].astype(jnp.float32)
                       + p_all[2, src_rows, :].astype(jnp.float32)
                       + p_all[3, src_rows, :].astype(jnp.float32))
                out_ref[pl.ds(256 * c + 64 * g, 64), :] = acc

        for r in sends:
            r.wait_send()

    out = pl.pallas_call(
        body,
        out_shape=jax.ShapeDtypeStruct((SQ, DMODEL), jnp.float32),
        in_specs=[
            pl.BlockSpec(memory_space=pltpu.VMEM),
            pl.BlockSpec(memory_space=pltpu.VMEM),
            pl.BlockSpec(memory_space=pltpu.MemorySpace.HBM),
            pl.BlockSpec(memory_space=pltpu.MemorySpace.HBM),
            pl.BlockSpec(memory_space=pltpu.VMEM),
        ],
        out_specs=pl.BlockSpec(memory_space=pltpu.VMEM),
        scratch_shapes=[
            pltpu.VMEM((N_DEV, SKV_LOCAL, HQ_LOCAL, DH), jnp.bfloat16),
            pltpu.VMEM((N_DEV, SKV_LOCAL, HQ_LOCAL, DH), jnp.bfloat16),
            pltpu.VMEM((N_DEV, SQ, DMODEL), jnp.bfloat16),
            pltpu.VMEM((SQ, DMODEL), jnp.bfloat16),
            pltpu.VMEM((SQ, DMODEL), jnp.bfloat16),
            pltpu.VMEM((32, 256, 1), jnp.float32),
            pltpu.VMEM((32, 256, 1), jnp.float32),
            pltpu.SemaphoreType.DMA((N_DEV,)),
            pltpu.SemaphoreType.DMA((N_DEV,)),
            pltpu.SemaphoreType.DMA((N_DEV, 4)),
            pltpu.SemaphoreType.DMA((N_DEV,)),
            pltpu.SemaphoreType.DMA((N_DEV,)),
            pltpu.SemaphoreType.DMA((N_DEV, 4)),
            pltpu.SemaphoreType.DMA((2,)),
        ],
        compiler_params=pltpu.CompilerParams(
            collective_id=0, vmem_limit_bytes=54 * 1024 * 1024),
    )(xb, wqb, kb, vb, wob)
    return out[None]
